# baseline (speedup 1.0000x reference)
import jax
import jax.numpy as jnp
from jax import lax
from jax.experimental import pallas as pl
from jax.experimental.pallas import tpu as pltpu
from jax.experimental.pallas import tpu_sc as plsc

K = 1024
D = 64
BETA = 0.25
P = 1024

NC, NS = 2, 16
NW = NC * NS
HN = 8 * P
PPW = HN // NW
CH = PPW // 128


def _dist_argmin_block(x_ref, emb_ref, idx_ref, loss_ref, se_ref):
    i = pl.program_id(0)
    x = x_ref[0]
    e = emb_ref[...]

    @pl.when(i == 0)
    def _init():
        se_ref[...] = jnp.sum(e ** 2, axis=1, keepdims=True)
        loss_ref[...] = jnp.zeros_like(loss_ref)

    m2 = lax.dot_general(e + e, x, (((1,), (0,)), ((), ())),
                         preferred_element_type=jnp.float32)
    sx = jnp.sum(x ** 2, axis=0, keepdims=True)
    dist = sx + se_ref[...] - m2
    minv = jnp.min(dist, axis=0, keepdims=True)
    ids = lax.broadcasted_iota(jnp.int32, (K, P), 0)
    idx = jnp.min(jnp.where(dist == minv, ids, K), axis=0)
    idx_ref[...] = idx.reshape(1, 1, P)
    loss_ref[...] += jnp.reshape(jnp.sum(minv), (1, 1))


def _tc_half(cols_half, embedding_weight):
    hb = cols_half.shape[0]
    return pl.pallas_call(
        _dist_argmin_block,
        grid=(hb,),
        in_specs=[pl.BlockSpec((1, D, P), lambda i: (i, 0, 0)),
                  pl.BlockSpec((K, D), lambda i: (0, 0))],
        out_specs=[pl.BlockSpec((1, 1, P), lambda i: (i, 0, 0)),
                   pl.BlockSpec((1, 1), lambda i: (0, 0))],
        out_shape=[jax.ShapeDtypeStruct((hb, 1, P), jnp.int32),
                   jax.ShapeDtypeStruct((1, 1), jnp.float32)],
        scratch_shapes=[pltpu.VMEM((K, 1), jnp.float32)],
    )(cols_half, embedding_weight)


def _sc_gather_body(emb_hbm, idx_hbm, out_hbm, idx_v, rows_v, sem):
    wid = lax.axis_index("s") * NC + lax.axis_index("c")
    pltpu.sync_copy(idx_hbm.at[wid], idx_v)
    copies = [
        pltpu.async_copy(emb_hbm.at[idx_v.at[j]],
                         rows_v.at[pl.ds(j * 128, 128)], sem)
        for j in range(CH)
    ]
    for c in copies:
        c.wait()
    pltpu.sync_copy(rows_v, out_hbm.at[wid])


_sc_gather = pl.kernel(
    _sc_gather_body,
    out_type=jax.ShapeDtypeStruct((NW, PPW, D), jnp.float32),
    mesh=plsc.VectorSubcoreMesh(core_axis_name="c", subcore_axis_name="s",
                                num_cores=NC, num_subcores=NS),
    scratch_types=[
        pltpu.VMEM((CH, 128), jnp.int32),
        pltpu.VMEM((PPW, D), jnp.float32),
        pltpu.SemaphoreType.DMA,
    ],
    compiler_params=pltpu.CompilerParams(use_tc_tiling_on_sc=False),
)


def kernel(latents, embedding_weight):
    b, c, h, w = latents.shape
    n = b * h * w
    cols = latents.reshape(b, c, h * w)
    hb = b // 2
    idx0, loss0 = _tc_half(cols[:hb], embedding_weight)
    idx1, loss1 = _tc_half(cols[hb:], embedding_weight)
    q0 = _sc_gather(embedding_weight, idx0.reshape(NW, CH, 128))
    q1 = _sc_gather(embedding_weight, idx1.reshape(NW, CH, 128))
    t0 = q0.reshape(hb, h * w, c).transpose(0, 2, 1)
    t1 = q1.reshape(hb, h * w, c).transpose(0, 2, 1)
    out = jnp.concatenate([t0, t1], axis=0).reshape(b, c, h, w)
    l = (loss0[0, 0] + loss1[0, 0]) / (n * D)
    return (out, l * BETA, l)


# speedup vs baseline: 1.5164x; 1.5164x over previous
import jax
import jax.numpy as jnp
from jax import lax
from jax.experimental import pallas as pl
from jax.experimental.pallas import tpu as pltpu

K = 1024
D = 64
BETA = 0.25
P = 1024
IB = 2


def _vq_block(x_ref, emb_ref, out_ref, loss_ref, se_ref):
    first = (pl.program_id(0) == 0) & (pl.program_id(1) == 0)
    e = emb_ref[...]

    @pl.when(first)
    def _init():
        se_ref[...] = jnp.sum(e ** 2, axis=1, keepdims=True)
        loss_ref[...] = jnp.zeros_like(loss_ref)

    e2 = e + e
    for sub in range(IB):
        x = x_ref[sub]
        m2 = lax.dot_general(e2, x, (((1,), (0,)), ((), ())),
                             preferred_element_type=jnp.float32)
        sx = jnp.sum(x ** 2, axis=0, keepdims=True)
        dist = sx + se_ref[...] - m2
        minv = jnp.min(dist, axis=0, keepdims=True)
        ids = lax.broadcasted_iota(jnp.int32, (K, P), 0)
        idx = jnp.min(jnp.where(dist == minv, ids, K), axis=0)
        oh = (ids == idx[None, :]).astype(jnp.float32)
        out_ref[sub] = lax.dot_general(e, oh, (((0,), (0,)), ((), ())),
                                       preferred_element_type=jnp.float32)
        loss_ref[...] += jnp.reshape(jnp.sum(minv), (1, 1))


def kernel(latents, embedding_weight):
    b, c, h, w = latents.shape
    n = b * h * w
    cols = latents.reshape(b, c, h * w)
    out_cols, loss = pl.pallas_call(
        _vq_block,
        grid=(b // IB, h * w // P),
        in_specs=[pl.BlockSpec((IB, D, P), lambda i, j: (i, 0, j)),
                  pl.BlockSpec((K, D), lambda i, j: (0, 0))],
        out_specs=[pl.BlockSpec((IB, D, P), lambda i, j: (i, 0, j)),
                   pl.BlockSpec((1, 1), lambda i, j: (0, 0))],
        out_shape=[jax.ShapeDtypeStruct((b, D, h * w), jnp.float32),
                   jax.ShapeDtypeStruct((1, 1), jnp.float32)],
        scratch_shapes=[pltpu.VMEM((K, 1), jnp.float32)],
    )(cols, embedding_weight)
    l = loss[0, 0] / (n * D)
    return (out_cols.reshape(b, c, h, w), l * BETA, l)
